# baseline (speedup 1.0000x reference)
import jax
import jax.numpy as jnp
from jax import lax
from jax.experimental import pallas as pl
from jax.experimental.pallas import tpu as pltpu
from jax.experimental.pallas import tpu_sc as plsc

_B = 64
_V = 1_000_000
_LANES = 16
_CW = 128
_NCHUNK = _B * _V // _CW
_VB = 16384
_NBLK = (_V + _VB - 1) // _VB
_LASTW = _V - (_NBLK - 1) * _VB


_TAIL = (_V // _CW) * _CW
_TAILW = _V - _TAIL


def _sc_gather_body(x_hbm, cls_hbm, out_hbm, tail_hbm, cls_v, rows_v, tail_v,
                    sem):
    wid = lax.axis_index("s") * 2 + lax.axis_index("c")

    @pl.when(wid < _B // _LANES)
    def _():
        base = wid * _LANES
        pltpu.sync_copy(cls_hbm.at[pl.ds(base, _LANES)], cls_v)
        starts = jnp.minimum((cls_v[...] >> 7) << 7, _TAIL - _CW)
        descs = []
        for i in range(_LANES):
            rstart = pl.multiple_of(base + (i // 8) * 8, 8)
            cstart = pl.multiple_of(starts[i], _CW)
            descs.append(pltpu.async_copy(
                x_hbm.at[pl.ds(rstart, 8), pl.ds(cstart, _CW)],
                rows_v.at[i], sem))
        for d in descs:
            d.wait()
        pltpu.sync_copy(rows_v, out_hbm.at[pl.ds(base, _LANES)])
        pltpu.sync_copy(x_hbm.at[pl.ds(base, _LANES), pl.ds(_TAIL, _TAILW)],
                        tail_v)
        pltpu.sync_copy(tail_v, tail_hbm.at[pl.ds(base, _LANES)])


def _sc_gather(x, cls):
    mesh = plsc.VectorSubcoreMesh(core_axis_name="c", subcore_axis_name="s")
    return pl.kernel(
        _sc_gather_body,
        mesh=mesh,
        out_type=(jax.ShapeDtypeStruct((_B, 8, _CW), jnp.float32),
                  jax.ShapeDtypeStruct((_B, _TAILW), jnp.float32)),
        scratch_types=[
            pltpu.VMEM((_LANES,), jnp.int32),
            pltpu.VMEM((_LANES, 8, _CW), jnp.float32),
            pltpu.VMEM((_LANES, _TAILW), jnp.float32),
            pltpu.SemaphoreType.DMA,
        ],
    )(x, cls)


def _count_body(rows_ref, tail_ref, cls_ref, x_ref, top1_ref, top5_ref,
                rank_ref, xc_ref, lane_ref, ones_ref):
    i = pl.program_id(0)
    c = cls_ref[...]

    @pl.when(i == 0)
    def _():
        rank_ref[...] = jnp.zeros_like(rank_ref)
        ones_ref[...] = jnp.ones_like(ones_ref)
        lane_ref[...] = lax.broadcasted_iota(jnp.int32, (_B, _VB), 1)
        c3 = c.reshape(_B, 1, 1)
        brow3 = lax.broadcasted_iota(jnp.int32, (_B, 8, _CW), 0)
        sub3 = lax.broadcasted_iota(jnp.int32, (_B, 8, _CW), 1)
        lane3 = lax.broadcasted_iota(jnp.int32, (_B, 8, _CW), 2)
        hot = (sub3 == (brow3 & 7)) & (lane3 == (c3 & (_CW - 1)))
        xc_slab = jnp.sum(jnp.where(hot, rows_ref[...], 0.0), axis=(1, 2))
        hot_t = lax.broadcasted_iota(jnp.int32, (_B, _TAILW), 1) == (c - _TAIL)
        xc_tail = jnp.sum(jnp.where(hot_t, tail_ref[...], 0.0), axis=1)
        xc_ref[...] = jnp.where(c[:, 0] >= _TAIL, xc_tail,
                                xc_slab).reshape(_B, 1)

    vals = x_ref[...]
    xc = xc_ref[...]
    lanes = lane_ref[...]
    eqb = (vals == xc) & (lanes < (c - i * _VB))

    def _accum(mask):
        contrib = jnp.where(mask, 1.0, 0.0)
        rank_ref[...] = rank_ref[...] + lax.dot_general(
            contrib, ones_ref[...], (((1,), (0,)), ((), ())),
            preferred_element_type=jnp.float32)

    @pl.when(i < _NBLK - 1)
    def _():
        _accum((vals > xc) | eqb)

    @pl.when(i == _NBLK - 1)
    def _():
        _accum(((vals > xc) & (lanes < _LASTW)) | eqb)
        rank = rank_ref[:, :1]
        top1_ref[...] = jnp.sum((rank == 0.0).astype(jnp.int32),
                                keepdims=True)
        top5_ref[...] = jnp.sum((rank < 5.0).astype(jnp.int32),
                                keepdims=True)


def _tc_count(x, rows, tail, cls):
    return pl.pallas_call(
        _count_body,
        grid=(_NBLK,),
        in_specs=[
            pl.BlockSpec((_B, 8, _CW), lambda i: (0, 0, 0)),
            pl.BlockSpec((_B, _TAILW), lambda i: (0, 0)),
            pl.BlockSpec((_B, 1), lambda i: (0, 0)),
            pl.BlockSpec((_B, _VB), lambda i: (0, i)),
        ],
        out_specs=[
            pl.BlockSpec((1, 1), lambda i: (0, 0)),
            pl.BlockSpec((1, 1), lambda i: (0, 0)),
        ],
        out_shape=[
            jax.ShapeDtypeStruct((1, 1), jnp.int32),
            jax.ShapeDtypeStruct((1, 1), jnp.int32),
        ],
        scratch_shapes=[
            pltpu.VMEM((_B, _CW), jnp.float32),
            pltpu.VMEM((_B, 1), jnp.float32),
            pltpu.VMEM((_B, _VB), jnp.int32),
            pltpu.VMEM((_VB, _CW), jnp.float32),
        ],
        compiler_params=pltpu.CompilerParams(
            dimension_semantics=("arbitrary",)),
    )(rows, tail, cls, x)


def kernel(x, classes):
    cls = classes.astype(jnp.int32).reshape(_B)
    rows, tail = _sc_gather(x, cls)
    top1, top5 = _tc_count(x, rows, tail, cls.reshape(_B, 1))
    return top1[0, 0], top5[0, 0]


# speedup vs baseline: 1.0141x; 1.0141x over previous
import jax
import jax.numpy as jnp
from jax import lax
from jax.experimental import pallas as pl
from jax.experimental.pallas import tpu as pltpu
from jax.experimental.pallas import tpu_sc as plsc

_B = 64
_V = 1_000_000
_LANES = 16
_CW = 128
_NCHUNK = _B * _V // _CW
_VB = 16384
_NBLK = (_V + _VB - 1) // _VB
_LASTW = _V - (_NBLK - 1) * _VB


_TAIL = (_V // _CW) * _CW
_TAILW = _V - _TAIL


def _sc_gather_body(x_hbm, cls_hbm, out_hbm, tail_hbm, cls_v, rows_v, tail_v,
                    sem):
    wid = lax.axis_index("s") * 2 + lax.axis_index("c")

    @pl.when(wid < _B // _LANES)
    def _():
        base = wid * _LANES
        pltpu.sync_copy(cls_hbm.at[pl.ds(base, _LANES)], cls_v)
        starts = jnp.minimum((cls_v[...] >> 7) << 7, _TAIL - _CW)
        descs = []
        for i in range(_LANES):
            rstart = pl.multiple_of(base + (i // 8) * 8, 8)
            cstart = pl.multiple_of(starts[i], _CW)
            descs.append(pltpu.async_copy(
                x_hbm.at[pl.ds(rstart, 8), pl.ds(cstart, _CW)],
                rows_v.at[i], sem))
        for d in descs:
            d.wait()
        pltpu.sync_copy(rows_v, out_hbm.at[pl.ds(base, _LANES)])
        pltpu.sync_copy(x_hbm.at[pl.ds(base, _LANES), pl.ds(_TAIL, _TAILW)],
                        tail_v)
        pltpu.sync_copy(tail_v, tail_hbm.at[pl.ds(base, _LANES)])


def _sc_gather(x, cls):
    mesh = plsc.VectorSubcoreMesh(core_axis_name="c", subcore_axis_name="s")
    return pl.kernel(
        _sc_gather_body,
        mesh=mesh,
        out_type=(jax.ShapeDtypeStruct((_B, 8, _CW), jnp.float32),
                  jax.ShapeDtypeStruct((_B, _TAILW), jnp.float32)),
        scratch_types=[
            pltpu.VMEM((_LANES,), jnp.int32),
            pltpu.VMEM((_LANES, 8, _CW), jnp.float32),
            pltpu.VMEM((_LANES, _TAILW), jnp.float32),
            pltpu.SemaphoreType.DMA,
        ],
    )(x, cls)


def _count_body(rows_ref, tail_ref, cls_ref, x_ref, top1_ref, top5_ref,
                rank_ref, xc_ref, lane_ref, ones_ref):
    i = pl.program_id(0)
    c = cls_ref[...]

    @pl.when(i == 0)
    def _():
        rank_ref[...] = jnp.zeros_like(rank_ref)
        ones_ref[...] = jnp.ones_like(ones_ref)
        lane_ref[...] = lax.broadcasted_iota(jnp.int32, (_B, _VB), 1)
        c3 = c.reshape(_B, 1, 1)
        brow3 = lax.broadcasted_iota(jnp.int32, (_B, 8, _CW), 0)
        sub3 = lax.broadcasted_iota(jnp.int32, (_B, 8, _CW), 1)
        lane3 = lax.broadcasted_iota(jnp.int32, (_B, 8, _CW), 2)
        hot = (sub3 == (brow3 & 7)) & (lane3 == (c3 & (_CW - 1)))
        xc_slab = jnp.sum(jnp.where(hot, rows_ref[...], 0.0), axis=(1, 2))
        hot_t = lax.broadcasted_iota(jnp.int32, (_B, _TAILW), 1) == (c - _TAIL)
        xc_tail = jnp.sum(jnp.where(hot_t, tail_ref[...], 0.0), axis=1)
        xc_ref[...] = jnp.where(c[:, 0] >= _TAIL, xc_tail,
                                xc_slab).reshape(_B, 1)

    vals = x_ref[...]
    xc = xc_ref[...]
    lanes = lane_ref[...]
    eqb = (vals == xc) & (lanes < (c - i * _VB))

    def _accum(mask):
        contrib = jnp.where(mask, 1.0, 0.0).astype(jnp.bfloat16)
        rank_ref[...] = rank_ref[...] + lax.dot_general(
            contrib, ones_ref[...], (((1,), (0,)), ((), ())),
            preferred_element_type=jnp.float32)

    @pl.when(i < _NBLK - 1)
    def _():
        _accum((vals > xc) | eqb)

    @pl.when(i == _NBLK - 1)
    def _():
        _accum(((vals > xc) & (lanes < _LASTW)) | eqb)
        rank = rank_ref[:, :1]
        top1_ref[...] = jnp.sum((rank == 0.0).astype(jnp.int32),
                                keepdims=True)
        top5_ref[...] = jnp.sum((rank < 5.0).astype(jnp.int32),
                                keepdims=True)


def _tc_count(x, rows, tail, cls):
    return pl.pallas_call(
        _count_body,
        grid=(_NBLK,),
        in_specs=[
            pl.BlockSpec((_B, 8, _CW), lambda i: (0, 0, 0)),
            pl.BlockSpec((_B, _TAILW), lambda i: (0, 0)),
            pl.BlockSpec((_B, 1), lambda i: (0, 0)),
            pl.BlockSpec((_B, _VB), lambda i: (0, i)),
        ],
        out_specs=[
            pl.BlockSpec((1, 1), lambda i: (0, 0)),
            pl.BlockSpec((1, 1), lambda i: (0, 0)),
        ],
        out_shape=[
            jax.ShapeDtypeStruct((1, 1), jnp.int32),
            jax.ShapeDtypeStruct((1, 1), jnp.int32),
        ],
        scratch_shapes=[
            pltpu.VMEM((_B, _CW), jnp.float32),
            pltpu.VMEM((_B, 1), jnp.float32),
            pltpu.VMEM((_B, _VB), jnp.int32),
            pltpu.VMEM((_VB, _CW), jnp.bfloat16),
        ],
        compiler_params=pltpu.CompilerParams(
            dimension_semantics=("arbitrary",)),
    )(rows, tail, cls, x)


def kernel(x, classes):
    cls = classes.astype(jnp.int32).reshape(_B)
    rows, tail = _sc_gather(x, cls)
    top1, top5 = _tc_count(x, rows, tail, cls.reshape(_B, 1))
    return top1[0, 0], top5[0, 0]


# speedup vs baseline: 1.1684x; 1.1521x over previous
import jax
import jax.numpy as jnp
from jax import lax
from jax.experimental import pallas as pl
from jax.experimental.pallas import tpu as pltpu
from jax.experimental.pallas import tpu_sc as plsc

_B = 64
_V = 1_000_000
_LANES = 16
_CW = 128
_NCHUNK = _B * _V // _CW
_VB = 16384
_NBLK = (_V + _VB - 1) // _VB
_LASTW = _V - (_NBLK - 1) * _VB


_TAIL = (_V // _CW) * _CW
_TAILW = _V - _TAIL


def _sc_gather_body(x_hbm, cls_hbm, out_hbm, tail_hbm, cls_v, rows_v, tail_v,
                    sem):
    wid = lax.axis_index("s") * 2 + lax.axis_index("c")

    @pl.when(wid < _B // _LANES)
    def _():
        base = wid * _LANES
        pltpu.sync_copy(cls_hbm.at[pl.ds(base, _LANES)], cls_v)
        starts = jnp.minimum((cls_v[...] >> 7) << 7, _TAIL - _CW)
        descs = []
        for i in range(_LANES):
            rstart = pl.multiple_of(base + (i // 8) * 8, 8)
            cstart = pl.multiple_of(starts[i], _CW)
            descs.append(pltpu.async_copy(
                x_hbm.at[pl.ds(rstart, 8), pl.ds(cstart, _CW)],
                rows_v.at[i], sem))
        for d in descs:
            d.wait()
        pltpu.sync_copy(rows_v, out_hbm.at[pl.ds(base, _LANES)])
        pltpu.sync_copy(x_hbm.at[pl.ds(base, _LANES), pl.ds(_TAIL, _TAILW)],
                        tail_v)
        pltpu.sync_copy(tail_v, tail_hbm.at[pl.ds(base, _LANES)])


def _sc_gather(x, cls):
    mesh = plsc.VectorSubcoreMesh(core_axis_name="c", subcore_axis_name="s")
    return pl.kernel(
        _sc_gather_body,
        mesh=mesh,
        out_type=(jax.ShapeDtypeStruct((_B, 8, _CW), jnp.float32),
                  jax.ShapeDtypeStruct((_B, _TAILW), jnp.float32)),
        scratch_types=[
            pltpu.VMEM((_LANES,), jnp.int32),
            pltpu.VMEM((_LANES, 8, _CW), jnp.float32),
            pltpu.VMEM((_LANES, _TAILW), jnp.float32),
            pltpu.SemaphoreType.DMA,
        ],
    )(x, cls)


def _count_body(rows_ref, tail_ref, cls_ref, x_ref, top1_ref, top5_ref,
                acc_ref, xc_ref, xclo_ref, lane_ref):
    i = pl.program_id(0)
    c = cls_ref[...]

    @pl.when(i == 0)
    def _():
        acc_ref[...] = jnp.zeros_like(acc_ref)
        lane_ref[...] = lax.broadcasted_iota(jnp.int32, (_B, _VB), 1)
        c3 = c.reshape(_B, 1, 1)
        brow3 = lax.broadcasted_iota(jnp.int32, (_B, 8, _CW), 0)
        sub3 = lax.broadcasted_iota(jnp.int32, (_B, 8, _CW), 1)
        lane3 = lax.broadcasted_iota(jnp.int32, (_B, 8, _CW), 2)
        hot = (sub3 == (brow3 & 7)) & (lane3 == (c3 & (_CW - 1)))
        xc_slab = jnp.sum(jnp.where(hot, rows_ref[...], 0.0), axis=(1, 2))
        hot_t = lax.broadcasted_iota(jnp.int32, (_B, _TAILW), 1) == (c - _TAIL)
        xc_tail = jnp.sum(jnp.where(hot_t, tail_ref[...], 0.0), axis=1)
        xc = jnp.where(c[:, 0] >= _TAIL, xc_tail, xc_slab).reshape(_B, 1)
        xc_ref[...] = xc
        b = lax.bitcast_convert_type(xc, jnp.int32)
        lo = jnp.where(b > 0, b - 1,
                       jnp.where(b == 0, jnp.int32(-2147483647), b + 1))
        xclo_ref[...] = lax.bitcast_convert_type(lo, jnp.float32)

    vals = x_ref[...]
    xc = xc_ref[...]
    xclo = xclo_ref[...]
    lanes = lane_ref[...]
    thr = jnp.where(lanes < (c - i * _VB), xclo, xc)

    @pl.when(i < _NBLK - 1)
    def _():
        acc_ref[...] = acc_ref[...] + (vals > thr).astype(jnp.int32)

    @pl.when(i == _NBLK - 1)
    def _():
        thr2 = jnp.where(lanes < _LASTW, thr, jnp.float32(jnp.inf))
        acc_ref[...] = acc_ref[...] + (vals > thr2).astype(jnp.int32)
        rank = jnp.sum(acc_ref[...], axis=1, keepdims=True)
        top1_ref[...] = jnp.sum((rank == 0).astype(jnp.int32), keepdims=True)
        top5_ref[...] = jnp.sum((rank < 5).astype(jnp.int32), keepdims=True)


def _tc_count(x, rows, tail, cls):
    return pl.pallas_call(
        _count_body,
        grid=(_NBLK,),
        in_specs=[
            pl.BlockSpec((_B, 8, _CW), lambda i: (0, 0, 0)),
            pl.BlockSpec((_B, _TAILW), lambda i: (0, 0)),
            pl.BlockSpec((_B, 1), lambda i: (0, 0)),
            pl.BlockSpec((_B, _VB), lambda i: (0, i)),
        ],
        out_specs=[
            pl.BlockSpec((1, 1), lambda i: (0, 0)),
            pl.BlockSpec((1, 1), lambda i: (0, 0)),
        ],
        out_shape=[
            jax.ShapeDtypeStruct((1, 1), jnp.int32),
            jax.ShapeDtypeStruct((1, 1), jnp.int32),
        ],
        scratch_shapes=[
            pltpu.VMEM((_B, _VB), jnp.int32),
            pltpu.VMEM((_B, 1), jnp.float32),
            pltpu.VMEM((_B, 1), jnp.float32),
            pltpu.VMEM((_B, _VB), jnp.int32),
        ],
        compiler_params=pltpu.CompilerParams(
            dimension_semantics=("arbitrary",)),
    )(rows, tail, cls, x)


def kernel(x, classes):
    cls = classes.astype(jnp.int32).reshape(_B)
    rows, tail = _sc_gather(x, cls)
    top1, top5 = _tc_count(x, rows, tail, cls.reshape(_B, 1))
    return top1[0, 0], top5[0, 0]


# speedup vs baseline: 1.1774x; 1.0077x over previous
import jax
import jax.numpy as jnp
from jax import lax
from jax.experimental import pallas as pl
from jax.experimental.pallas import tpu as pltpu
from jax.experimental.pallas import tpu_sc as plsc

_B = 64
_V = 1_000_000
_LANES = 16
_CW = 128
_NCHUNK = _B * _V // _CW
_VB = 16384
_NBLK = (_V + _VB - 1) // _VB
_LASTW = _V - (_NBLK - 1) * _VB


_TAIL = (_V // _CW) * _CW
_TAILW = _V - _TAIL


def _sc_gather_body(x_hbm, cls_hbm, out_hbm, tail_hbm, cls_v, rows_v, tail_v,
                    sem):
    wid = lax.axis_index("s") * 2 + lax.axis_index("c")

    @pl.when(wid < _B // _LANES)
    def _():
        base = wid * _LANES
        pltpu.sync_copy(cls_hbm.at[pl.ds(base, _LANES)], cls_v)
        starts = jnp.minimum((cls_v[...] >> 7) << 7, _TAIL - _CW)
        descs = []
        for i in range(_LANES):
            rstart = pl.multiple_of(base + (i // 8) * 8, 8)
            cstart = pl.multiple_of(starts[i], _CW)
            descs.append(pltpu.async_copy(
                x_hbm.at[pl.ds(rstart, 8), pl.ds(cstart, _CW)],
                rows_v.at[i], sem))
        for d in descs:
            d.wait()
        pltpu.sync_copy(rows_v, out_hbm.at[pl.ds(base, _LANES)])
        pltpu.sync_copy(x_hbm.at[pl.ds(base, _LANES), pl.ds(_TAIL, _TAILW)],
                        tail_v)
        pltpu.sync_copy(tail_v, tail_hbm.at[pl.ds(base, _LANES)])


def _sc_gather(x, cls):
    mesh = plsc.VectorSubcoreMesh(core_axis_name="c", subcore_axis_name="s")
    return pl.kernel(
        _sc_gather_body,
        mesh=mesh,
        out_type=(jax.ShapeDtypeStruct((_B, 8, _CW), jnp.float32),
                  jax.ShapeDtypeStruct((_B, _TAILW), jnp.float32)),
        scratch_types=[
            pltpu.VMEM((_LANES,), jnp.int32),
            pltpu.VMEM((_LANES, 8, _CW), jnp.float32),
            pltpu.VMEM((_LANES, _TAILW), jnp.float32),
            pltpu.SemaphoreType.DMA,
        ],
    )(x, cls)


def _count_body(rows_ref, tail_ref, cls_ref, x_ref, top1_ref, top5_ref,
                acc_ref, xc_ref, xclo_ref, lane_ref):
    i = pl.program_id(0)
    c = cls_ref[...]

    @pl.when(i == 0)
    def _():
        acc_ref[...] = jnp.zeros_like(acc_ref)
        lane_ref[...] = lax.broadcasted_iota(jnp.int32, (_B, _VB), 1)
        c3 = c.reshape(_B, 1, 1)
        brow3 = lax.broadcasted_iota(jnp.int32, (_B, 8, _CW), 0)
        sub3 = lax.broadcasted_iota(jnp.int32, (_B, 8, _CW), 1)
        lane3 = lax.broadcasted_iota(jnp.int32, (_B, 8, _CW), 2)
        hot = (sub3 == (brow3 & 7)) & (lane3 == (c3 & (_CW - 1)))
        xc_slab = jnp.sum(jnp.where(hot, rows_ref[...], 0.0), axis=(1, 2))
        hot_t = lax.broadcasted_iota(jnp.int32, (_B, _TAILW), 1) == (c - _TAIL)
        xc_tail = jnp.sum(jnp.where(hot_t, tail_ref[...], 0.0), axis=1)
        xc = jnp.where(c[:, 0] >= _TAIL, xc_tail, xc_slab).reshape(_B, 1)
        xc_ref[...] = xc
        b = lax.bitcast_convert_type(xc, jnp.int32)
        lo = jnp.where(b > 0, b - 1,
                       jnp.where(b == 0, jnp.int32(-2147483647), b + 1))
        xclo_ref[...] = lax.bitcast_convert_type(lo, jnp.float32)

    vals = x_ref[...]
    xc = xc_ref[...]
    xclo = xclo_ref[...]
    lanes = lane_ref[...]
    thr = jnp.where(lanes < (c - i * _VB), xclo, xc)

    def _accum(mask):
        s = jnp.where(mask, 1.0, 0.0)
        w = _VB
        while w > _CW:
            w //= 2
            s = s[:, :w] + s[:, w:2 * w]
        acc_ref[...] = acc_ref[...] + s

    @pl.when(i < _NBLK - 1)
    def _():
        _accum(vals > thr)

    @pl.when(i == _NBLK - 1)
    def _():
        thr2 = jnp.where(lanes < _LASTW, thr, jnp.float32(jnp.inf))
        _accum(vals > thr2)
        rank = jnp.sum(acc_ref[...], axis=1, keepdims=True)
        top1_ref[...] = jnp.sum((rank == 0).astype(jnp.int32), keepdims=True)
        top5_ref[...] = jnp.sum((rank < 5).astype(jnp.int32), keepdims=True)


def _tc_count(x, rows, tail, cls):
    return pl.pallas_call(
        _count_body,
        grid=(_NBLK,),
        in_specs=[
            pl.BlockSpec((_B, 8, _CW), lambda i: (0, 0, 0)),
            pl.BlockSpec((_B, _TAILW), lambda i: (0, 0)),
            pl.BlockSpec((_B, 1), lambda i: (0, 0)),
            pl.BlockSpec((_B, _VB), lambda i: (0, i)),
        ],
        out_specs=[
            pl.BlockSpec((1, 1), lambda i: (0, 0)),
            pl.BlockSpec((1, 1), lambda i: (0, 0)),
        ],
        out_shape=[
            jax.ShapeDtypeStruct((1, 1), jnp.int32),
            jax.ShapeDtypeStruct((1, 1), jnp.int32),
        ],
        scratch_shapes=[
            pltpu.VMEM((_B, _CW), jnp.float32),
            pltpu.VMEM((_B, 1), jnp.float32),
            pltpu.VMEM((_B, 1), jnp.float32),
            pltpu.VMEM((_B, _VB), jnp.int32),
        ],
        compiler_params=pltpu.CompilerParams(
            dimension_semantics=("arbitrary",)),
    )(rows, tail, cls, x)


def kernel(x, classes):
    cls = classes.astype(jnp.int32).reshape(_B)
    rows, tail = _sc_gather(x, cls)
    top1, top5 = _tc_count(x, rows, tail, cls.reshape(_B, 1))
    return top1[0, 0], top5[0, 0]


# speedup vs baseline: 1.2895x; 1.0952x over previous
import jax
import jax.numpy as jnp
from jax import lax
from jax.experimental import pallas as pl
from jax.experimental.pallas import tpu as pltpu
from jax.experimental.pallas import tpu_sc as plsc

_B = 64
_V = 1_000_000
_LANES = 16
_CW = 128
_NCHUNK = _B * _V // _CW
_VB = 32768
_NBLK = (_V + _VB - 1) // _VB
_LASTW = _V - (_NBLK - 1) * _VB


_TAIL = (_V // _CW) * _CW
_TAILW = _V - _TAIL


def _sc_gather_body(x_hbm, cls_hbm, out_hbm, tail_hbm, cls_v, rows_v, tail_v,
                    sem):
    wid = lax.axis_index("s") * 2 + lax.axis_index("c")

    @pl.when(wid < _B // _LANES)
    def _():
        base = wid * _LANES
        pltpu.sync_copy(cls_hbm.at[pl.ds(base, _LANES)], cls_v)
        starts = jnp.minimum((cls_v[...] >> 7) << 7, _TAIL - _CW)
        descs = []
        for i in range(_LANES):
            rstart = pl.multiple_of(base + (i // 8) * 8, 8)
            cstart = pl.multiple_of(starts[i], _CW)
            descs.append(pltpu.async_copy(
                x_hbm.at[pl.ds(rstart, 8), pl.ds(cstart, _CW)],
                rows_v.at[i], sem))
        for d in descs:
            d.wait()
        pltpu.sync_copy(rows_v, out_hbm.at[pl.ds(base, _LANES)])
        pltpu.sync_copy(x_hbm.at[pl.ds(base, _LANES), pl.ds(_TAIL, _TAILW)],
                        tail_v)
        pltpu.sync_copy(tail_v, tail_hbm.at[pl.ds(base, _LANES)])


def _sc_gather(x, cls):
    mesh = plsc.VectorSubcoreMesh(core_axis_name="c", subcore_axis_name="s")
    return pl.kernel(
        _sc_gather_body,
        mesh=mesh,
        out_type=(jax.ShapeDtypeStruct((_B, 8, _CW), jnp.float32),
                  jax.ShapeDtypeStruct((_B, _TAILW), jnp.float32)),
        scratch_types=[
            pltpu.VMEM((_LANES,), jnp.int32),
            pltpu.VMEM((_LANES, 8, _CW), jnp.float32),
            pltpu.VMEM((_LANES, _TAILW), jnp.float32),
            pltpu.SemaphoreType.DMA,
        ],
    )(x, cls)


def _count_body(rows_ref, tail_ref, cls_ref, x_ref, top1_ref, top5_ref,
                acc_ref, xc_ref, xclo_ref, lane_ref):
    i = pl.program_id(0)
    c = cls_ref[...]

    @pl.when(i == 0)
    def _():
        acc_ref[...] = jnp.zeros_like(acc_ref)
        lane_ref[...] = lax.broadcasted_iota(jnp.int32, (_B, _VB), 1)
        c3 = c.reshape(_B, 1, 1)
        brow3 = lax.broadcasted_iota(jnp.int32, (_B, 8, _CW), 0)
        sub3 = lax.broadcasted_iota(jnp.int32, (_B, 8, _CW), 1)
        lane3 = lax.broadcasted_iota(jnp.int32, (_B, 8, _CW), 2)
        hot = (sub3 == (brow3 & 7)) & (lane3 == (c3 & (_CW - 1)))
        xc_slab = jnp.sum(jnp.where(hot, rows_ref[...], 0.0), axis=(1, 2))
        hot_t = lax.broadcasted_iota(jnp.int32, (_B, _TAILW), 1) == (c - _TAIL)
        xc_tail = jnp.sum(jnp.where(hot_t, tail_ref[...], 0.0), axis=1)
        xc = jnp.where(c[:, 0] >= _TAIL, xc_tail, xc_slab).reshape(_B, 1)
        xc_ref[...] = xc
        b = lax.bitcast_convert_type(xc, jnp.int32)
        lo = jnp.where(b > 0, b - 1,
                       jnp.where(b == 0, jnp.int32(-2147483647), b + 1))
        xclo_ref[...] = lax.bitcast_convert_type(lo, jnp.float32)

    vals = x_ref[...]
    xc = xc_ref[...]
    xclo = xclo_ref[...]
    lanes = lane_ref[...]
    thr = jnp.where(lanes < (c - i * _VB), xclo, xc)

    def _accum(mask):
        s = jnp.where(mask, 1.0, 0.0)
        w = _VB
        while w > _CW:
            w //= 2
            s = s[:, :w] + s[:, w:2 * w]
        acc_ref[...] = acc_ref[...] + s

    @pl.when(i < _NBLK - 1)
    def _():
        _accum(vals > thr)

    @pl.when(i == _NBLK - 1)
    def _():
        thr2 = jnp.where(lanes < _LASTW, thr, jnp.float32(jnp.inf))
        _accum(vals > thr2)
        rank = jnp.sum(acc_ref[...], axis=1, keepdims=True)
        top1_ref[...] = jnp.sum((rank == 0).astype(jnp.int32), keepdims=True)
        top5_ref[...] = jnp.sum((rank < 5).astype(jnp.int32), keepdims=True)


def _tc_count(x, rows, tail, cls):
    return pl.pallas_call(
        _count_body,
        grid=(_NBLK,),
        in_specs=[
            pl.BlockSpec((_B, 8, _CW), lambda i: (0, 0, 0)),
            pl.BlockSpec((_B, _TAILW), lambda i: (0, 0)),
            pl.BlockSpec((_B, 1), lambda i: (0, 0)),
            pl.BlockSpec((_B, _VB), lambda i: (0, i)),
        ],
        out_specs=[
            pl.BlockSpec((1, 1), lambda i: (0, 0)),
            pl.BlockSpec((1, 1), lambda i: (0, 0)),
        ],
        out_shape=[
            jax.ShapeDtypeStruct((1, 1), jnp.int32),
            jax.ShapeDtypeStruct((1, 1), jnp.int32),
        ],
        scratch_shapes=[
            pltpu.VMEM((_B, _CW), jnp.float32),
            pltpu.VMEM((_B, 1), jnp.float32),
            pltpu.VMEM((_B, 1), jnp.float32),
            pltpu.VMEM((_B, _VB), jnp.int32),
        ],
        compiler_params=pltpu.CompilerParams(
            dimension_semantics=("arbitrary",)),
    )(rows, tail, cls, x)


def kernel(x, classes):
    cls = classes.astype(jnp.int32).reshape(_B)
    rows, tail = _sc_gather(x, cls)
    top1, top5 = _tc_count(x, rows, tail, cls.reshape(_B, 1))
    return top1[0, 0], top5[0, 0]
